# baseline (speedup 1.0000x reference)
import functools

import jax
import jax.numpy as jnp
import numpy as np
from jax.experimental import pallas as pl
from jax.experimental.pallas import tpu as pltpu

_K = 32
_BM = 256

_INT32_MIN = -2147483648


def _sae_block_kernel(x_ref, we_ref, be_ref, wd_ref, bp_ref, o_ref):
    bp = bp_ref[...]
    xc = x_ref[...] - bp
    pre = jax.lax.dot_general(
        xc, we_ref[...], (((1,), (1,)), ((), ())),
        preferred_element_type=jnp.float32,
        precision=jax.lax.Precision.DEFAULT,
    ) + be_ref[...]

    bm, lat = pre.shape

    def fwd(f):
        iv = jax.lax.bitcast_convert_type(f, jnp.int32)
        return iv ^ ((iv >> 31) & 0x7FFFFFFF)

    def inv(sv):
        iv = sv ^ ((sv >> 31) & 0x7FFFFFFF)
        return jax.lax.bitcast_convert_type(iv, jnp.float32)

    nchunk = 128
    m = pre[:, :nchunk]
    for c in range(1, lat // nchunk):
        m = jnp.maximum(m, pre[:, c * nchunk:(c + 1) * nchunk])

    lo0 = fwd(jnp.min(m, axis=1, keepdims=True))
    hi0 = fwd(jnp.max(m, axis=1, keepdims=True)) + 1

    def count_ge(thr):
        parts = [(pre[:, j * 128:(j + 1) * 128] >= thr).astype(jnp.int32)
                 for j in range(lat // 128)]
        while len(parts) > 1:
            parts = [parts[i] + parts[i + 1] for i in range(0, len(parts), 2)]
        return jnp.sum(parts[0], axis=1, keepdims=True)

    def cond(st):
        b, alldone, lo, hi, done, tau = st
        return (b < 40) & (alldone == 0)

    def bodyw(st):
        b, alldone, lo, hi, done, tau = st
        w3 = jnp.maximum(jax.lax.div(hi - lo, jnp.int32(3)), 1)
        t1 = lo + w3
        t2 = hi - w3
        c1 = count_ge(inv(t1))
        c2 = count_ge(inv(t2))
        notdone = done == 0
        exact1 = (c1 == _K) & notdone
        exact2 = (c2 == _K) & notdone
        stuck = (hi - lo <= 1) & notdone
        tau = jnp.where(exact2, t2,
                        jnp.where(exact1, t1, jnp.where(stuck, lo, tau)))
        ndone_b = exact1 | exact2 | stuck
        keep = ndone_b | (done != 0)
        ge2 = c2 >= _K
        ge1 = c1 >= _K
        nlo = jnp.where(keep, lo, jnp.where(ge2, t2, jnp.where(ge1, t1, lo)))
        nhi = jnp.where(keep, hi, jnp.where(ge2, hi, jnp.where(ge1, t2, t1)))
        ndone = jnp.where(ndone_b, jnp.int32(1), done)
        return (b + 1, jnp.min(ndone), nlo, nhi, ndone, tau)

    st0 = (jnp.int32(0), jnp.int32(0), lo0, hi0,
           jnp.zeros((bm, 1), jnp.int32), lo0)
    _, _, _, _, _, tau = jax.lax.while_loop(cond, bodyw, st0)

    z = jnp.where(pre >= inv(tau), pre, 0.0)
    rec = jax.lax.dot_general(
        z, wd_ref[...], (((1,), (1,)), ((), ())),
        preferred_element_type=jnp.float32,
        precision=jax.lax.Precision.DEFAULT,
    )
    o_ref[...] = rec + bp


def kernel(x, W_enc, b_enc, W_dec, b_pre, model_idx):
    n, d = x.shape
    latent = W_enc.shape[0]
    assert n % _BM == 0
    be2 = b_enc.reshape(1, latent)
    bp2 = b_pre.reshape(1, d)
    return pl.pallas_call(
        _sae_block_kernel,
        grid=(n // _BM,),
        in_specs=[
            pl.BlockSpec((_BM, d), lambda i: (i, 0)),
            pl.BlockSpec((latent, d), lambda i: (0, 0)),
            pl.BlockSpec((1, latent), lambda i: (0, 0)),
            pl.BlockSpec((d, latent), lambda i: (0, 0)),
            pl.BlockSpec((1, d), lambda i: (0, 0)),
        ],
        out_specs=pl.BlockSpec((_BM, d), lambda i: (i, 0)),
        out_shape=jax.ShapeDtypeStruct((n, d), jnp.float32),
        compiler_params=pltpu.CompilerParams(
            dimension_semantics=("parallel",),
        ),
    )(x, W_enc, be2, W_dec, bp2)


# speedup vs baseline: 1.1534x; 1.1534x over previous
import functools

import jax
import jax.numpy as jnp
from jax.experimental import pallas as pl
from jax.experimental.pallas import tpu as pltpu

_K = 32
_BM = 256


def _sae_block_kernel(x_ref, we_ref, be_ref, wd_ref, bp_ref, o_ref):
    bp = bp_ref[...]
    xc = x_ref[...] - bp
    pre = jax.lax.dot_general(
        xc, we_ref[...], (((1,), (1,)), ((), ())),
        preferred_element_type=jnp.float32,
        precision=jax.lax.Precision.DEFAULT,
    ) + be_ref[...]

    bm, lat = pre.shape

    def fwd(f):
        iv = jax.lax.bitcast_convert_type(f, jnp.int32)
        return iv ^ ((iv >> 31) & 0x7FFFFFFF)

    def inv(sv):
        iv = sv ^ ((sv >> 31) & 0x7FFFFFFF)
        return jax.lax.bitcast_convert_type(iv, jnp.float32)

    nchunk = 128
    m = pre[:, :nchunk]
    for c in range(1, lat // nchunk):
        m = jnp.maximum(m, pre[:, c * nchunk:(c + 1) * nchunk])

    lo0 = fwd(jnp.min(m, axis=1, keepdims=True))
    hi0 = fwd(jnp.max(m, axis=1, keepdims=True)) + 1

    def count_ge(thr):
        parts = [(pre[:, j * 128:(j + 1) * 128] >= thr).astype(jnp.int32)
                 for j in range(lat // 128)]
        while len(parts) > 1:
            parts = [parts[i] + parts[i + 1] for i in range(0, len(parts), 2)]
        return jnp.sum(parts[0], axis=1, keepdims=True)

    def cond(st):
        b, alldone, lo, hi, done, tau = st
        return (b < 40) & (alldone == 0)

    def step(lo, hi, done, tau):
        mid = lo + jax.lax.shift_right_logical(hi - lo, 1)
        cnt = count_ge(inv(mid))
        notdone = done == 0
        exact = (cnt == _K) & notdone
        stuck = (hi - lo <= 1) & notdone
        tau = jnp.where(exact, mid, jnp.where(stuck, lo, tau))
        ndone_b = exact | stuck
        ge = cnt >= _K
        keep = ndone_b | (done != 0)
        nlo = jnp.where(keep | jnp.logical_not(ge), lo, mid)
        nhi = jnp.where(keep | ge, hi, mid)
        ndone = jnp.where(ndone_b, jnp.int32(1), done)
        return nlo, nhi, ndone, tau

    def bodyw(st):
        b, alldone, lo, hi, done, tau = st
        lo, hi, done, tau = step(lo, hi, done, tau)
        lo, hi, done, tau = step(lo, hi, done, tau)
        return (b + 1, jnp.min(done), lo, hi, done, tau)

    st0 = (jnp.int32(0), jnp.int32(0), lo0, hi0,
           jnp.zeros((bm, 1), jnp.int32), lo0)
    _, _, _, _, _, tau = jax.lax.while_loop(cond, bodyw, st0)

    z = jnp.where(pre >= inv(tau), pre, 0.0)
    rec = jax.lax.dot_general(
        z, wd_ref[...], (((1,), (1,)), ((), ())),
        preferred_element_type=jnp.float32,
        precision=jax.lax.Precision.DEFAULT,
    )
    o_ref[...] = rec + bp


def kernel(x, W_enc, b_enc, W_dec, b_pre, model_idx):
    n, d = x.shape
    latent = W_enc.shape[0]
    assert n % _BM == 0
    be2 = b_enc.reshape(1, latent)
    bp2 = b_pre.reshape(1, d)
    return pl.pallas_call(
        _sae_block_kernel,
        grid=(n // _BM,),
        in_specs=[
            pl.BlockSpec((_BM, d), lambda i: (i, 0)),
            pl.BlockSpec((latent, d), lambda i: (0, 0)),
            pl.BlockSpec((1, latent), lambda i: (0, 0)),
            pl.BlockSpec((d, latent), lambda i: (0, 0)),
            pl.BlockSpec((1, d), lambda i: (0, 0)),
        ],
        out_specs=pl.BlockSpec((_BM, d), lambda i: (i, 0)),
        out_shape=jax.ShapeDtypeStruct((n, d), jnp.float32),
        compiler_params=pltpu.CompilerParams(
            dimension_semantics=("parallel",),
        ),
    )(x, W_enc, be2, W_dec, bp2)
